# Initial kernel scaffold; baseline (speedup 1.0000x reference)
import functools

import jax
import jax.numpy as jnp
from jax import lax
from jax.experimental import pallas as pl
from jax.experimental.pallas import tpu as pltpu
from jax.experimental.pallas import tpu_sc as plsc

N = 10000
E = 320000
D_IN = 128
H1 = 64
D_OUT = 2

NC = 2
NS = 16
NW = NC * NS
CH = 128
N_PAD = 10240
E_PAD = 327680
EPT = E_PAD // NW
ITERS = EPT // CH
ROWS_PT = N_PAD // NS

_mesh = plsc.VectorSubcoreMesh(core_axis_name="c", subcore_axis_name="s")


@functools.partial(
    pl.kernel,
    out_type=(
        jax.ShapeDtypeStruct((NC * N_PAD,), jnp.float32),
        jax.ShapeDtypeStruct((NC * N_PAD,), jnp.float32),
    ),
    mesh=_mesh,
    scratch_types=[
        pltpu.VMEM_SHARED((N_PAD,), jnp.float32),
        pltpu.VMEM_SHARED((N_PAD,), jnp.float32),
        pltpu.VMEM((CH,), jnp.int32),
        pltpu.VMEM((CH,), jnp.int32),
        pltpu.VMEM((CH,), jnp.float32),
    ],
)
def _degree_kernel(src_hbm, dst_hbm, zeros_hbm, sdeg_out, ddeg_out,
                   shist, dhist, srcb, dstb, onesb):
    c = lax.axis_index("c")
    s = lax.axis_index("s")
    base = (c * NS + s) * EPT

    for i in range(CH // 16):
        onesb[pl.ds(i * 16, 16)] = jnp.ones((16,), jnp.float32)

    pltpu.sync_copy(zeros_hbm.at[pl.ds(s * ROWS_PT, ROWS_PT)],
                    shist.at[pl.ds(s * ROWS_PT, ROWS_PT)])
    pltpu.sync_copy(zeros_hbm.at[pl.ds(s * ROWS_PT, ROWS_PT)],
                    dhist.at[pl.ds(s * ROWS_PT, ROWS_PT)])
    plsc.subcore_barrier()

    def body(g, carry):
        off = base + g * CH
        pltpu.sync_copy(src_hbm.at[pl.ds(off, CH)], srcb)
        pltpu.sync_copy(dst_hbm.at[pl.ds(off, CH)], dstb)
        pltpu.sync_copy(onesb, shist.at[srcb], add=True)
        pltpu.sync_copy(onesb, dhist.at[dstb], add=True)
        return carry

    lax.fori_loop(0, ITERS, body, 0)
    plsc.subcore_barrier()

    pltpu.sync_copy(shist.at[pl.ds(s * ROWS_PT, ROWS_PT)],
                    sdeg_out.at[pl.ds(c * N_PAD + s * ROWS_PT, ROWS_PT)])
    pltpu.sync_copy(dhist.at[pl.ds(s * ROWS_PT, ROWS_PT)],
                    ddeg_out.at[pl.ds(c * N_PAD + s * ROWS_PT, ROWS_PT)])


def _h_body(x_ref, deg_ref, w_ref, h_ref):
    deg = jnp.maximum(deg_ref[0] + deg_ref[1], 1.0)
    xs = x_ref[...] * lax.rsqrt(deg)[:, None]
    h_ref[...] = jnp.dot(xs, w_ref[...], preferred_element_type=jnp.float32)


_B_ROWS = 1280


def _stage2(x_pad, sdeg2, W1):
    return pl.pallas_call(
        _h_body,
        grid=(N_PAD // _B_ROWS,),
        in_specs=[
            pl.BlockSpec((_B_ROWS, D_IN), lambda i: (i, 0)),
            pl.BlockSpec((NC, _B_ROWS), lambda i: (0, i)),
            pl.BlockSpec((D_IN, H1), lambda i: (0, 0)),
        ],
        out_specs=pl.BlockSpec((_B_ROWS, H1), lambda i: (i, 0)),
        out_shape=jax.ShapeDtypeStruct((N_PAD, H1), jnp.float32),
    )(x_pad, sdeg2, W1)


@functools.partial(
    pl.kernel,
    out_type=jax.ShapeDtypeStruct((NC * N_PAD, H1), jnp.float32),
    mesh=_mesh,
    scratch_types=[
        pltpu.VMEM_SHARED((N_PAD, H1), jnp.float32),
        pltpu.VMEM((CH,), jnp.int32),
        pltpu.VMEM((CH,), jnp.int32),
        pltpu.VMEM((CH, H1), jnp.float32),
        pltpu.SemaphoreType.DMA,
    ],
)
def _scatter_kernel(h_hbm, src_hbm, dst_hbm, zeros_hbm, agg_out,
                    agg_sh, srcb, dstb, rows, gsem):
    c = lax.axis_index("c")
    s = lax.axis_index("s")
    base = (c * NS + s) * EPT

    pltpu.sync_copy(zeros_hbm, agg_sh.at[pl.ds(s * ROWS_PT, ROWS_PT)])
    plsc.subcore_barrier()

    def body(g, carry):
        off = base + g * CH
        pltpu.sync_copy(src_hbm.at[pl.ds(off, CH)], srcb)
        pltpu.sync_copy(dst_hbm.at[pl.ds(off, CH)], dstb)
        pltpu.async_copy(h_hbm.at[srcb], rows, gsem).wait()
        pltpu.sync_copy(rows, agg_sh.at[dstb], add=True)
        return carry

    lax.fori_loop(0, ITERS, body, 0)
    plsc.subcore_barrier()

    pltpu.sync_copy(agg_sh.at[pl.ds(s * ROWS_PT, ROWS_PT)],
                    agg_out.at[pl.ds(c * N_PAD + s * ROWS_PT, ROWS_PT)])


def _out_body(agg_ref, deg_ref, b1_ref, g1_ref, be1_ref, w2_ref, b2_ref, o_ref):
    eps = 1e-5
    a = agg_ref[0] + agg_ref[1]
    idn = lax.rsqrt(jnp.maximum(deg_ref[0] + deg_ref[1], 1.0))
    t = a * idn[:, None] + b1_ref[...]
    t = t * (g1_ref[...] * (1.0 / jnp.sqrt(1.0 + eps))) + be1_ref[...]
    t = jnp.maximum(t, 0.0)
    o_ref[...] = (jnp.dot(t, w2_ref[...], preferred_element_type=jnp.float32)
                  + b2_ref[...])


def _stage4(agg2, ddeg2, b1, gamma1, beta1, W2, b2):
    return pl.pallas_call(
        _out_body,
        grid=(N_PAD // _B_ROWS,),
        in_specs=[
            pl.BlockSpec((NC, _B_ROWS, H1), lambda i: (0, i, 0)),
            pl.BlockSpec((NC, _B_ROWS), lambda i: (0, i)),
            pl.BlockSpec((1, H1), lambda i: (0, 0)),
            pl.BlockSpec((1, H1), lambda i: (0, 0)),
            pl.BlockSpec((1, H1), lambda i: (0, 0)),
            pl.BlockSpec((H1, D_OUT), lambda i: (0, 0)),
            pl.BlockSpec((1, D_OUT), lambda i: (0, 0)),
        ],
        out_specs=pl.BlockSpec((_B_ROWS, D_OUT), lambda i: (i, 0)),
        out_shape=jax.ShapeDtypeStruct((N_PAD, D_OUT), jnp.float32),
    )(agg2, ddeg2, b1.reshape(1, H1), gamma1.reshape(1, H1),
      beta1.reshape(1, H1), W2, b2.reshape(1, D_OUT))


def kernel(x, edge_index, W1, b1, gamma1, beta1, W2, b2):
    src = edge_index[0]
    dst = edge_index[1]
    pad = jnp.int32(N) + (jnp.arange(E_PAD - E, dtype=jnp.int32) % (N_PAD - N))
    src_p = jnp.concatenate([src, pad])
    dst_p = jnp.concatenate([dst, pad])
    x_pad = jnp.pad(x, ((0, N_PAD - N), (0, 0)))
    zeros1 = jnp.zeros((N_PAD,), jnp.float32)
    zeros2 = jnp.zeros((ROWS_PT, H1), jnp.float32)

    sdeg, ddeg = _degree_kernel(src_p, dst_p, zeros1)
    sdeg2 = sdeg.reshape(NC, N_PAD)
    ddeg2 = ddeg.reshape(NC, N_PAD)

    h = _stage2(x_pad, sdeg2, W1)
    agg = _scatter_kernel(h, src_p, dst_p, zeros2)
    agg2 = agg.reshape(NC, N_PAD, H1)

    out = _stage4(agg2, ddeg2, b1, gamma1, beta1, W2, b2)
    return out[:N]


# speedup vs baseline: 6.5277x; 6.5277x over previous
import functools

import jax
import jax.numpy as jnp
from jax import lax
from jax.experimental import pallas as pl
from jax.experimental.pallas import tpu as pltpu
from jax.experimental.pallas import tpu_sc as plsc

N = 10000
E = 320000
D_IN = 128
H1 = 64
D_OUT = 2

NC = 2
NS = 16
NW = NC * NS
CH = 128
N_PAD = 10240
E_PAD = 327680
EPT = E_PAD // NW
ITERS = EPT // CH
ROWS_PT = N_PAD // NS

_mesh = plsc.VectorSubcoreMesh(core_axis_name="c", subcore_axis_name="s")


@functools.partial(
    pl.kernel,
    out_type=(
        jax.ShapeDtypeStruct((NC * N_PAD,), jnp.float32),
        jax.ShapeDtypeStruct((NC * N_PAD,), jnp.float32),
    ),
    mesh=_mesh,
    compiler_params=pltpu.CompilerParams(use_tc_tiling_on_sc=False),
    scratch_types=[
        pltpu.VMEM_SHARED((N_PAD,), jnp.float32),
        pltpu.VMEM_SHARED((N_PAD,), jnp.float32),
        pltpu.VMEM((CH,), jnp.int32),
        pltpu.VMEM((CH,), jnp.int32),
        pltpu.VMEM((CH,), jnp.float32),
    ],
)
def _degree_kernel(src_hbm, dst_hbm, zeros_hbm, sdeg_out, ddeg_out,
                   shist, dhist, srcb, dstb, onesb):
    c = lax.axis_index("c")
    s = lax.axis_index("s")
    base = (c * NS + s) * EPT

    for i in range(CH // 16):
        onesb[pl.ds(i * 16, 16)] = jnp.ones((16,), jnp.float32)

    pltpu.sync_copy(zeros_hbm.at[pl.ds(s * ROWS_PT, ROWS_PT)],
                    shist.at[pl.ds(s * ROWS_PT, ROWS_PT)])
    pltpu.sync_copy(zeros_hbm.at[pl.ds(s * ROWS_PT, ROWS_PT)],
                    dhist.at[pl.ds(s * ROWS_PT, ROWS_PT)])
    plsc.subcore_barrier()

    def body(g, carry):
        off = base + g * CH
        pltpu.sync_copy(src_hbm.at[pl.ds(off, CH)], srcb)
        pltpu.sync_copy(dst_hbm.at[pl.ds(off, CH)], dstb)
        pltpu.sync_copy(onesb, shist.at[srcb], add=True)
        pltpu.sync_copy(onesb, dhist.at[dstb], add=True)
        return carry

    lax.fori_loop(0, ITERS, body, 0)
    plsc.subcore_barrier()

    pltpu.sync_copy(shist.at[pl.ds(s * ROWS_PT, ROWS_PT)],
                    sdeg_out.at[pl.ds(c * N_PAD + s * ROWS_PT, ROWS_PT)])
    pltpu.sync_copy(dhist.at[pl.ds(s * ROWS_PT, ROWS_PT)],
                    ddeg_out.at[pl.ds(c * N_PAD + s * ROWS_PT, ROWS_PT)])


def _h_body(x_ref, deg_ref, w_ref, h_ref):
    deg = jnp.maximum(deg_ref[0] + deg_ref[1], 1.0)
    xs = x_ref[...] * lax.rsqrt(deg)[:, None]
    h_ref[...] = jnp.dot(xs, w_ref[...], preferred_element_type=jnp.float32)


_B_ROWS = 1280


def _stage2(x_pad, sdeg2, W1):
    return pl.pallas_call(
        _h_body,
        grid=(N_PAD // _B_ROWS,),
        in_specs=[
            pl.BlockSpec((_B_ROWS, D_IN), lambda i: (i, 0)),
            pl.BlockSpec((NC, _B_ROWS), lambda i: (0, i)),
            pl.BlockSpec((D_IN, H1), lambda i: (0, 0)),
        ],
        out_specs=pl.BlockSpec((_B_ROWS, H1), lambda i: (i, 0)),
        out_shape=jax.ShapeDtypeStruct((N_PAD, H1), jnp.float32),
    )(x_pad, sdeg2, W1)


@functools.partial(
    pl.kernel,
    out_type=jax.ShapeDtypeStruct((NC * N_PAD, H1), jnp.float32),
    mesh=_mesh,
    compiler_params=pltpu.CompilerParams(use_tc_tiling_on_sc=False),
    scratch_types=[
        pltpu.VMEM_SHARED((N_PAD, H1), jnp.float32),
        pltpu.VMEM((CH,), jnp.int32),
        pltpu.VMEM((CH,), jnp.int32),
        pltpu.VMEM((CH, H1), jnp.float32),
        pltpu.SemaphoreType.DMA,
    ],
)
def _scatter_kernel(h_hbm, src_hbm, dst_hbm, zeros_hbm, agg_out,
                    agg_sh, srcb, dstb, rows, gsem):
    c = lax.axis_index("c")
    s = lax.axis_index("s")
    base = (c * NS + s) * EPT

    pltpu.sync_copy(zeros_hbm, agg_sh.at[pl.ds(s * ROWS_PT, ROWS_PT)])
    plsc.subcore_barrier()

    def body(g, carry):
        off = base + g * CH
        pltpu.sync_copy(src_hbm.at[pl.ds(off, CH)], srcb)
        pltpu.sync_copy(dst_hbm.at[pl.ds(off, CH)], dstb)
        pltpu.async_copy(h_hbm.at[srcb], rows, gsem).wait()
        pltpu.sync_copy(rows, agg_sh.at[dstb], add=True)
        return carry

    lax.fori_loop(0, ITERS, body, 0)
    plsc.subcore_barrier()

    pltpu.sync_copy(agg_sh.at[pl.ds(s * ROWS_PT, ROWS_PT)],
                    agg_out.at[pl.ds(c * N_PAD + s * ROWS_PT, ROWS_PT)])


def _out_body(agg_ref, deg_ref, b1_ref, g1_ref, be1_ref, w2_ref, b2_ref, o_ref):
    eps = 1e-5
    a = agg_ref[0] + agg_ref[1]
    idn = lax.rsqrt(jnp.maximum(deg_ref[0] + deg_ref[1], 1.0))
    t = a * idn[:, None] + b1_ref[...]
    t = t * (g1_ref[...] * (1.0 / jnp.sqrt(1.0 + eps))) + be1_ref[...]
    t = jnp.maximum(t, 0.0)
    o_ref[...] = (jnp.dot(t, w2_ref[...], preferred_element_type=jnp.float32)
                  + b2_ref[...])


def _stage4(agg2, ddeg2, b1, gamma1, beta1, W2, b2):
    return pl.pallas_call(
        _out_body,
        grid=(N_PAD // _B_ROWS,),
        in_specs=[
            pl.BlockSpec((NC, _B_ROWS, H1), lambda i: (0, i, 0)),
            pl.BlockSpec((NC, _B_ROWS), lambda i: (0, i)),
            pl.BlockSpec((1, H1), lambda i: (0, 0)),
            pl.BlockSpec((1, H1), lambda i: (0, 0)),
            pl.BlockSpec((1, H1), lambda i: (0, 0)),
            pl.BlockSpec((H1, D_OUT), lambda i: (0, 0)),
            pl.BlockSpec((1, D_OUT), lambda i: (0, 0)),
        ],
        out_specs=pl.BlockSpec((_B_ROWS, D_OUT), lambda i: (i, 0)),
        out_shape=jax.ShapeDtypeStruct((N_PAD, D_OUT), jnp.float32),
    )(agg2, ddeg2, b1.reshape(1, H1), gamma1.reshape(1, H1),
      beta1.reshape(1, H1), W2, b2.reshape(1, D_OUT))


def kernel(x, edge_index, W1, b1, gamma1, beta1, W2, b2):
    src = edge_index[0]
    dst = edge_index[1]
    pad = jnp.int32(N) + (jnp.arange(E_PAD - E, dtype=jnp.int32) % (N_PAD - N))
    src_p = jnp.concatenate([src, pad])
    dst_p = jnp.concatenate([dst, pad])
    x_pad = jnp.pad(x, ((0, N_PAD - N), (0, 0)))
    zeros1 = jnp.zeros((N_PAD,), jnp.float32)
    zeros2 = jnp.zeros((ROWS_PT, H1), jnp.float32)

    sdeg, ddeg = _degree_kernel(src_p, dst_p, zeros1)
    sdeg2 = sdeg.reshape(NC, N_PAD)
    ddeg2 = ddeg.reshape(NC, N_PAD)

    h = _stage2(x_pad, sdeg2, W1)
    agg = _scatter_kernel(h, src_p, dst_p, zeros2)
    agg2 = agg.reshape(NC, N_PAD, H1)

    out = _stage4(agg2, ddeg2, b1, gamma1, beta1, W2, b2)
    return out[:N]
